# baseline (speedup 1.0000x reference)
import functools

import jax
import jax.numpy as jnp
from jax import lax
from jax.experimental import pallas as pl
from jax.experimental.pallas import tpu as pltpu
from jax.experimental.pallas import tpu_sc as plsc

N = 10000
E = 320000
D = 128

NC = 2
NS = 16
NW = NC * NS
L = 16

EPT = E // NW
EPTP = 10240
BLK1 = 1024
NB1 = EPTP // BLK1
EP = NW * EPTP
BLK2 = 2048
NB2 = EP // BLK2
CPT = D // NW
ZW = N * CPT


def _tc_prep_body(x_ref, w_ref, al_ref, ar_ref, z_ref, el_ref, er_ref,
                  em_ref):
    z = jnp.dot(x_ref[...], w_ref[...], preferred_element_type=jnp.float32)
    z_ref[...] = z
    el = jnp.sum(z * al_ref[...], axis=1, keepdims=True)
    el_ref[...] = el
    er_ref[...] = jnp.sum(z * ar_ref[...], axis=1, keepdims=True)
    em_ref[...] = jnp.full((1, L), jnp.max(el), jnp.float32)


def _tc_prep(x, w, al, ar):
    return pl.pallas_call(
        _tc_prep_body,
        out_shape=(
            jax.ShapeDtypeStruct((N, D), jnp.float32),
            jax.ShapeDtypeStruct((N, 1), jnp.float32),
            jax.ShapeDtypeStruct((N, 1), jnp.float32),
            jax.ShapeDtypeStruct((1, L), jnp.float32),
        ),
    )(x, w, al, ar)


def _tc_mid_body(acc_ref, asum_ref, b_ref, g_ref, be_ref, w_ref, al_ref,
                 ar_ref, z_ref, el_ref, er_ref, em_ref):
    s = jnp.sum(asum_ref[...], axis=0)
    h = acc_ref[...] / (s + 1e-9) + b_ref[...]
    mu = jnp.mean(h, axis=1, keepdims=True)
    var = jnp.mean((h - mu) ** 2, axis=1, keepdims=True)
    h = (h - mu) / jnp.sqrt(var + 1e-5) * g_ref[...] + be_ref[...]
    h = jnp.maximum(h, 0.0)
    z = jnp.dot(h, w_ref[...], preferred_element_type=jnp.float32)
    z_ref[...] = z
    el = jnp.sum(z * al_ref[...], axis=1, keepdims=True)
    el_ref[...] = el
    er_ref[...] = jnp.sum(z * ar_ref[...], axis=1, keepdims=True)
    em_ref[...] = jnp.full((1, L), jnp.max(el), jnp.float32)


def _tc_mid(acc, asum3, b, gamma, beta, w, al, ar):
    return pl.pallas_call(
        _tc_mid_body,
        out_shape=(
            jax.ShapeDtypeStruct((N, D), jnp.float32),
            jax.ShapeDtypeStruct((N, 1), jnp.float32),
            jax.ShapeDtypeStruct((N, 1), jnp.float32),
            jax.ShapeDtypeStruct((1, L), jnp.float32),
        ),
    )(acc, asum3, b, gamma, beta, w, al, ar)


def _tc_final_body(acc_ref, asum_ref, b_ref, out_ref):
    s = jnp.sum(asum_ref[...], axis=0)
    out_ref[...] = acc_ref[...] / (s + 1e-9) + b_ref[...]


def _tc_final(acc, asum3, b):
    return pl.pallas_call(
        _tc_final_body,
        out_shape=jax.ShapeDtypeStruct((N, D), jnp.float32),
    )(acc, asum3, b)


_MESH = plsc.VectorSubcoreMesh(core_axis_name="c", subcore_axis_name="s",
                               num_cores=NC, num_subcores=NS)
_CP = pltpu.CompilerParams(needs_layout_passes=False)


def _sc_w_body(src_hbm, dst_hbm, el_hbm, er_hbm, emax_hbm, zerosn_hbm,
               wtab_out, asum_out,
               el_v, er_v, emax_s, asumstg,
               srcb0, srcb1, dstb0, dstb1, wsc0, wsc1,
               asum_sh, wsem0, wsem1):
    cid = lax.axis_index("c")
    sid = lax.axis_index("s")
    wid = cid * NS + sid
    ebase = wid * EPTP

    pltpu.sync_copy(emax_hbm, emax_s)
    pltpu.sync_copy(el_hbm, el_v)
    pltpu.sync_copy(er_hbm, er_v)

    @pl.when(sid == 1)
    def _zero_asum():
        pltpu.sync_copy(zerosn_hbm, asum_sh)
    plsc.subcore_barrier()

    emaxv = emax_s[pl.ds(0, L)]
    lane = lax.iota(jnp.int32, L)
    srcbs = (srcb0, srcb1)
    dstbs = (dstb0, dstb1)
    wscs = (wsc0, wsc1)
    wsems = (wsem0, wsem1)

    def wait_w_scatter(bp):
        pltpu.make_async_copy(wscs[bp], asum_sh.at[dstbs[bp]],
                              wsems[bp]).wait()

    def _pair(i, carry):
        for bp in (0, 1):
            b = 2 * i + bp

            @pl.when(b >= 2)
            def _ww(bp=bp):
                wait_w_scatter(bp)

            pltpu.sync_copy(src_hbm.at[pl.ds(ebase + b * BLK1, BLK1)],
                            srcbs[bp])
            pltpu.sync_copy(dst_hbm.at[pl.ds(ebase + b * BLK1, BLK1)],
                            dstbs[bp])

            def _vec(v, c2, bp=bp, b=b):
                sl = pl.ds(v * L, L)
                s = srcbs[bp][sl]
                d = dstbs[bp][sl]
                els = plsc.load_gather(el_v, [s])
                erd = plsc.load_gather(er_v, [d])
                u = els + erd
                ev = jnp.maximum(u, 0.2 * u)
                t = erd + emaxv
                cv = jnp.maximum(t, 0.2 * t)
                w = jnp.exp(ev - cv)
                pos = lane + (b * BLK1 + v * L)
                w = jnp.where(pos < EPT, w, 0.0)
                wscs[bp][sl] = w
                return c2
            lax.fori_loop(0, BLK1 // L, _vec, 0)

            pltpu.async_copy(wscs[bp], asum_sh.at[dstbs[bp]], wsems[bp],
                             add=True)
            pltpu.sync_copy(wscs[bp],
                            wtab_out.at[pl.ds(ebase + b * BLK1, BLK1)])
        return carry
    lax.fori_loop(0, NB1 // 2, _pair, 0)

    wait_w_scatter(0)
    wait_w_scatter(1)
    plsc.subcore_barrier()

    @pl.when(sid == 0)
    def _wb():
        def _wbk(k5, c3):
            pltpu.sync_copy(asum_sh.at[pl.ds(k5 * 1000, 1000)], asumstg)
            pltpu.sync_copy(asumstg,
                            asum_out.at[pl.ds(cid * N + k5 * 1000, 1000)])
            return c3
        lax.fori_loop(0, N // 1000, _wbk, 0)


_sc_w = functools.partial(
    pl.kernel,
    out_type=(
        jax.ShapeDtypeStruct((EP,), jnp.float32),
        jax.ShapeDtypeStruct((NC * N,), jnp.float32),
    ),
    mesh=_MESH,
    compiler_params=_CP,
    scratch_types=[
        pltpu.VMEM((N,), jnp.float32),
        pltpu.VMEM((N,), jnp.float32),
        pltpu.VMEM((L,), jnp.float32),
        pltpu.VMEM((1000,), jnp.float32),
        pltpu.VMEM((BLK1,), jnp.int32),
        pltpu.VMEM((BLK1,), jnp.int32),
        pltpu.VMEM((BLK1,), jnp.int32),
        pltpu.VMEM((BLK1,), jnp.int32),
        pltpu.VMEM((BLK1,), jnp.float32),
        pltpu.VMEM((BLK1,), jnp.float32),
        pltpu.VMEM_SHARED((N,), jnp.float32),
        pltpu.SemaphoreType.DMA,
        pltpu.SemaphoreType.DMA,
    ],
)(_sc_w_body)


def _sc_apply_body(src_hbm, dst_hbm, wtab_hbm, zp_hbm,
                   accp_out,
                   zcols, acccols,
                   srcb0, srcb1, dstb0, dstb1, wb0, wb1,
                   psem0, psem1):
    cid = lax.axis_index("c")
    sid = lax.axis_index("s")
    wid = cid * NS + sid
    pbase = wid * ZW

    pltpu.sync_copy(zp_hbm.at[pl.ds(pbase, ZW)], zcols)

    zero16 = jnp.zeros((L,), jnp.float32)

    def _za(i, carry):
        acccols[pl.ds(i * L, L)] = zero16
        return carry
    lax.fori_loop(0, ZW // L, _za, 0)

    srcbs = (srcb0, srcb1)
    dstbs = (dstb0, dstb1)
    wbs = (wb0, wb1)
    psems = (psem0, psem1)

    def issue_block(blk, p):
        off = blk * BLK2
        pltpu.async_copy(src_hbm.at[pl.ds(off, BLK2)], srcbs[p], psems[p])
        pltpu.async_copy(dst_hbm.at[pl.ds(off, BLK2)], dstbs[p], psems[p])
        pltpu.async_copy(wtab_hbm.at[pl.ds(off, BLK2)], wbs[p], psems[p])

    def wait_block(blk, p):
        off = blk * BLK2
        pltpu.make_async_copy(src_hbm.at[pl.ds(off, BLK2)], srcbs[p],
                              psems[p]).wait()
        pltpu.make_async_copy(dst_hbm.at[pl.ds(off, BLK2)], dstbs[p],
                              psems[p]).wait()
        pltpu.make_async_copy(wtab_hbm.at[pl.ds(off, BLK2)], wbs[p],
                              psems[p]).wait()

    issue_block(0, 0)

    def _pair(i, carry):
        for p in (0, 1):
            blk = 2 * i + p
            q = 1 - p

            @pl.when(blk + 1 < NB2)
            def _ib(blk=blk, q=q):
                issue_block(blk + 1, q)

            wait_block(blk, p)

            def _grp(j, c2, p=p):
                sl = pl.ds(j * L, L)
                s4 = srcbs[p][sl] * CPT
                d4 = dstbs[p][sl] * CPT
                w = wbs[p][sl]
                for c in range(CPT):
                    zv = plsc.load_gather(zcols, [s4 + c])
                    plsc.addupdate_scatter(acccols, [d4 + c], zv * w)
                return c2
            lax.fori_loop(0, BLK2 // L, _grp, 0)
        return carry
    lax.fori_loop(0, NB2 // 2, _pair, 0)

    pltpu.sync_copy(acccols, accp_out.at[pl.ds(pbase, ZW)])


_sc_apply = functools.partial(
    pl.kernel,
    out_type=jax.ShapeDtypeStruct((NW * ZW,), jnp.float32),
    mesh=_MESH,
    compiler_params=_CP,
    scratch_types=[
        pltpu.VMEM((ZW,), jnp.float32),
        pltpu.VMEM((ZW,), jnp.float32),
        pltpu.VMEM((BLK2,), jnp.int32),
        pltpu.VMEM((BLK2,), jnp.int32),
        pltpu.VMEM((BLK2,), jnp.int32),
        pltpu.VMEM((BLK2,), jnp.int32),
        pltpu.VMEM((BLK2,), jnp.float32),
        pltpu.VMEM((BLK2,), jnp.float32),
        pltpu.SemaphoreType.DMA,
        pltpu.SemaphoreType.DMA,
    ],
)(_sc_apply_body)


def _layer(src1, dst1, z, el, er, em, zerosn):
    wtab, asum = _sc_w(src1, dst1, el.reshape(N), er.reshape(N),
                       em.reshape(L), zerosn)
    zp = z.reshape(N, NW, CPT).transpose(1, 0, 2).reshape(NW * ZW)
    accp = _sc_apply(src1, dst1, wtab, zp)
    acc = accp.reshape(NW, N, CPT).transpose(1, 0, 2).reshape(N, D)
    return acc, asum.reshape(NC, N, 1)


def kernel(feat, edge_index, W0, al0, ar0, b0, gamma, beta, W1, al1, ar1, b1):
    ei = edge_index.reshape(2, NW, EPT)
    eip = jnp.pad(ei, ((0, 0), (0, 0), (0, EPTP - EPT)))
    src1 = eip[0].reshape(EP)
    dst1 = eip[1].reshape(EP)
    zerosn = jnp.zeros((N,), jnp.float32)

    z0, el0, er0, em0 = _tc_prep(feat, W0, al0, ar0)
    acc0, asum0 = _layer(src1, dst1, z0, el0, er0, em0, zerosn)
    z1, el1, er1, em1 = _tc_mid(acc0, asum0, b0.reshape(1, D),
                                gamma.reshape(1, D), beta.reshape(1, D),
                                W1, al1, ar1)
    acc1, asum1 = _layer(src1, dst1, z1, el1, er1, em1, zerosn)
    return _tc_final(acc1, asum1, b1.reshape(1, D))


# speedup vs baseline: 2.7130x; 2.7130x over previous
import functools

import jax
import jax.numpy as jnp
from jax import lax
from jax.experimental import pallas as pl
from jax.experimental.pallas import tpu as pltpu
from jax.experimental.pallas import tpu_sc as plsc

N = 10000
E = 320000
D = 128

NC = 2
NS = 16
NW = NC * NS
L = 16

EPT = E // NW
K = 128
KH = K // 2
EPTP = 10240
BLK = 8
BLKE = BLK * K
NCH = EPTP // K
NBLK = NCH // BLK


def _tc_prep_body(x_ref, w_ref, al_ref, ar_ref, z_ref, el_ref, er_ref,
                  em_ref):
    z = jnp.dot(x_ref[...], w_ref[...], preferred_element_type=jnp.float32)
    z_ref[...] = z
    el = jnp.sum(z * al_ref[...], axis=1, keepdims=True)
    el_ref[...] = el
    er_ref[...] = jnp.sum(z * ar_ref[...], axis=1, keepdims=True)
    em_ref[...] = jnp.full((1, L), jnp.max(el), jnp.float32)


def _tc_prep(x, w, al, ar):
    return pl.pallas_call(
        _tc_prep_body,
        out_shape=(
            jax.ShapeDtypeStruct((N, D), jnp.float32),
            jax.ShapeDtypeStruct((N, 1), jnp.float32),
            jax.ShapeDtypeStruct((N, 1), jnp.float32),
            jax.ShapeDtypeStruct((1, L), jnp.float32),
        ),
    )(x, w, al, ar)


def _tc_mid_body(acc_ref, asum_ref, b_ref, g_ref, be_ref, w_ref, al_ref,
                 ar_ref, z_ref, el_ref, er_ref, em_ref):
    s = jnp.sum(asum_ref[...], axis=0)
    h = (acc_ref[0] + acc_ref[1]) / (s + 1e-9) + b_ref[...]
    mu = jnp.mean(h, axis=1, keepdims=True)
    var = jnp.mean((h - mu) ** 2, axis=1, keepdims=True)
    h = (h - mu) / jnp.sqrt(var + 1e-5) * g_ref[...] + be_ref[...]
    h = jnp.maximum(h, 0.0)
    z = jnp.dot(h, w_ref[...], preferred_element_type=jnp.float32)
    z_ref[...] = z
    el = jnp.sum(z * al_ref[...], axis=1, keepdims=True)
    el_ref[...] = el
    er_ref[...] = jnp.sum(z * ar_ref[...], axis=1, keepdims=True)
    em_ref[...] = jnp.full((1, L), jnp.max(el), jnp.float32)


def _tc_mid(acc, asum3, b, gamma, beta, w, al, ar):
    return pl.pallas_call(
        _tc_mid_body,
        out_shape=(
            jax.ShapeDtypeStruct((N, D), jnp.float32),
            jax.ShapeDtypeStruct((N, 1), jnp.float32),
            jax.ShapeDtypeStruct((N, 1), jnp.float32),
            jax.ShapeDtypeStruct((1, L), jnp.float32),
        ),
    )(acc, asum3, b, gamma, beta, w, al, ar)


def _tc_final_body(acc_ref, asum_ref, b_ref, out_ref):
    s = jnp.sum(asum_ref[...], axis=0)
    out_ref[...] = (acc_ref[0] + acc_ref[1]) / (s + 1e-9) + b_ref[...]


def _tc_final(acc, asum3, b):
    return pl.pallas_call(
        _tc_final_body,
        out_shape=jax.ShapeDtypeStruct((N, D), jnp.float32),
    )(acc, asum3, b)


def _sc_edge_body(src_hbm, dst4_hbm, el_hbm, er_hbm, emax_hbm, z_hbm,
                  zeros_hbm, zerosn_hbm,
                  acc_out, asum_out,
                  srcb, dstb, elb, erb, wbuf, rowbuf, emax_s, asumstg,
                  acc_sh, asum_sh, gsem0, gsem1, ssem0, ssem1):
    cid = lax.axis_index("c")
    sid = lax.axis_index("s")
    wid = cid * NS + sid
    ebase = wid * EPTP

    pltpu.sync_copy(emax_hbm, emax_s)

    @pl.when(sid == 0)
    def _zero_acc():
        pltpu.sync_copy(zeros_hbm, acc_sh)

    @pl.when(sid == 1)
    def _zero_asum():
        pltpu.sync_copy(zerosn_hbm, asum_sh)
    plsc.subcore_barrier()

    emaxv = emax_s[pl.ds(0, L)]
    lane = lax.iota(jnp.int32, L)
    gsems = (gsem0, gsem1)
    ssems = (ssem0, ssem1)

    def stage_block(b):
        bp = b % 2
        pltpu.sync_copy(src_hbm.at[pl.ds(ebase + b * BLKE, BLKE)],
                        srcb.at[bp])
        pltpu.sync_copy(dst4_hbm.at[wid, b], dstb.at[bp])

    def src_idx(cc, h):
        bp = (cc // BLK) % 2
        return srcb.at[bp, pl.ds((cc % BLK) * K + h * KH, KH)]

    def src_idx_full(cc):
        bp = (cc // BLK) % 2
        return srcb.at[bp, pl.ds((cc % BLK) * K, K)]

    def dst_idx(cc):
        bp = (cc // BLK) % 2
        return dstb.at[bp, cc % BLK]

    def issue_gathers(cc, p):
        pltpu.async_copy(z_hbm.at[src_idx(cc, 0)],
                         rowbuf.at[p, pl.ds(0, KH)], gsems[p])
        pltpu.async_copy(z_hbm.at[src_idx(cc, 1)],
                         rowbuf.at[p, pl.ds(KH, KH)], gsems[p])
        pltpu.async_copy(el_hbm.at[src_idx_full(cc)], elb.at[p], gsems[p])
        pltpu.async_copy(er_hbm.at[dst_idx(cc)], erb.at[p], gsems[p])

    def wait_gathers(cc, p):
        pltpu.make_async_copy(z_hbm.at[src_idx(cc, 0)],
                              rowbuf.at[p, pl.ds(0, KH)], gsems[p]).wait()
        pltpu.make_async_copy(z_hbm.at[src_idx(cc, 1)],
                              rowbuf.at[p, pl.ds(KH, KH)], gsems[p]).wait()
        pltpu.make_async_copy(el_hbm.at[src_idx_full(cc)], elb.at[p],
                              gsems[p]).wait()
        pltpu.make_async_copy(er_hbm.at[dst_idx(cc)], erb.at[p],
                              gsems[p]).wait()

    def issue_scatters(cc, p):
        pltpu.async_copy(wbuf.at[p], asum_sh.at[dst_idx(cc)], ssems[p],
                         add=True)
        pltpu.async_copy(rowbuf.at[p], acc_sh.at[dst_idx(cc)], ssems[p],
                         add=True)

    def wait_scatters(cc, p):
        pltpu.make_async_copy(wbuf.at[p], asum_sh.at[dst_idx(cc)],
                              ssems[p]).wait()
        pltpu.make_async_copy(rowbuf.at[p], acc_sh.at[dst_idx(cc)],
                              ssems[p]).wait()

    def compute_scale(cc, p):
        offv = cc * K
        for v in range(K // L):
            els = elb[p, pl.ds(v * L, L)]
            erd = erb[p, pl.ds(v * L, L)]
            u = els + erd
            ev = jnp.maximum(u, 0.2 * u)
            t = erd + emaxv
            cv = jnp.maximum(t, 0.2 * t)
            w = jnp.exp(ev - cv)
            pos = lane + (offv + v * L)
            w = jnp.where(pos < EPT, w, 0.0)
            wbuf[p, pl.ds(v * L, L)] = w

        def _svb(vb, c2):
            wv = wbuf[p, pl.ds(vb * L, L)]
            for l in range(L):
                j = vb * L + l
                wj = wv[l]
                for k in range(D // L):
                    sl = pl.ds(k * L, L)
                    rowbuf[p, j, sl] = rowbuf[p, j, sl] * wj
            return c2
        lax.fori_loop(0, K // L, _svb, 0)

    stage_block(0)
    issue_gathers(0, 0)

    def _pair(i, carry):
        for half in (0, 1):
            cc = 2 * i + half
            p = half
            q = 1 - half

            @pl.when(cc >= 1)
            def _ws(cc=cc, q=q):
                wait_scatters(cc - 1, q)

            @pl.when(jnp.logical_and(cc + 1 < NCH, (cc + 1) % BLK == 0))
            def _sb(cc=cc):
                stage_block((cc + 1) // BLK)

            @pl.when(cc + 1 < NCH)
            def _ig(cc=cc, q=q):
                issue_gathers(cc + 1, q)

            wait_gathers(cc, p)
            compute_scale(cc, p)
            issue_scatters(cc, p)
        return carry
    lax.fori_loop(0, NCH // 2, _pair, 0)

    wait_scatters(NCH - 1, 1)
    plsc.subcore_barrier()

    @pl.when(sid == 0)
    def _wb():
        def _wbk(k5, c3):
            pltpu.sync_copy(asum_sh.at[pl.ds(k5 * 1000, 1000)], asumstg)
            pltpu.sync_copy(asumstg,
                            asum_out.at[pl.ds(cid * N + k5 * 1000, 1000)])
            return c3
        lax.fori_loop(0, N // 1000, _wbk, 0)
        pltpu.sync_copy(acc_sh, acc_out.at[cid])


_sc_edge = functools.partial(
    pl.kernel,
    out_type=(
        jax.ShapeDtypeStruct((NC, N, D), jnp.float32),
        jax.ShapeDtypeStruct((NC * N,), jnp.float32),
    ),
    mesh=plsc.VectorSubcoreMesh(core_axis_name="c", subcore_axis_name="s",
                                num_cores=NC, num_subcores=NS),
    compiler_params=pltpu.CompilerParams(needs_layout_passes=False),
    scratch_types=[
        pltpu.VMEM((2, BLKE), jnp.int32),
        pltpu.VMEM((2, BLK, K), jnp.int32),
        pltpu.VMEM((2, K), jnp.float32),
        pltpu.VMEM((2, K), jnp.float32),
        pltpu.VMEM((2, K), jnp.float32),
        pltpu.VMEM((2, K, D), jnp.float32),
        pltpu.VMEM((L,), jnp.float32),
        pltpu.VMEM((1000,), jnp.float32),
        pltpu.VMEM_SHARED((N, D), jnp.float32),
        pltpu.VMEM_SHARED((N,), jnp.float32),
        pltpu.SemaphoreType.DMA,
        pltpu.SemaphoreType.DMA,
        pltpu.SemaphoreType.DMA,
        pltpu.SemaphoreType.DMA,
    ],
)(_sc_edge_body)


def kernel(feat, edge_index, W0, al0, ar0, b0, gamma, beta, W1, al1, ar1, b1):
    ei = edge_index.reshape(2, NW, EPT)
    eip = jnp.pad(ei, ((0, 0), (0, 0), (0, EPTP - EPT)))
    src = eip[0].reshape(NW * EPTP)
    dst4 = eip[1].reshape(NW, NBLK, BLK, K)
    zeros = jnp.zeros((N, D), jnp.float32)
    zerosn = jnp.zeros((N,), jnp.float32)

    z0, el0, er0, em0 = _tc_prep(feat, W0, al0, ar0)
    acc0, asum0 = _sc_edge(src, dst4, el0.reshape(N), er0.reshape(N),
                           em0.reshape(L), z0, zeros, zerosn)
    z1, el1, er1, em1 = _tc_mid(acc0, asum0.reshape(NC, N, 1),
                                b0.reshape(1, D), gamma.reshape(1, D),
                                beta.reshape(1, D), W1, al1, ar1)
    acc1, asum1 = _sc_edge(src, dst4, el1.reshape(N), er1.reshape(N),
                           em1.reshape(L), z1, zeros, zerosn)
    return _tc_final(acc1, asum1.reshape(NC, N, 1), b1.reshape(1, D))


# speedup vs baseline: 2.7146x; 1.0006x over previous
import functools

import jax
import jax.numpy as jnp
from jax import lax
from jax.experimental import pallas as pl
from jax.experimental.pallas import tpu as pltpu
from jax.experimental.pallas import tpu_sc as plsc

N = 10000
E = 320000
D = 128

NC = 2
NS = 16
NW = NC * NS
L = 16

EPT = E // NW
K = 128
EPTP = 10240
BLK = 8
BLKE = BLK * K
NCH = EPTP // K
NBLK = NCH // BLK


def _tc_prep_body(x_ref, w_ref, al_ref, ar_ref, z_ref, el_ref, er_ref,
                  em_ref):
    z = jnp.dot(x_ref[...], w_ref[...], preferred_element_type=jnp.float32)
    z_ref[...] = z
    el = jnp.sum(z * al_ref[...], axis=1, keepdims=True)
    el_ref[...] = el
    er_ref[...] = jnp.sum(z * ar_ref[...], axis=1, keepdims=True)
    em_ref[...] = jnp.full((1, L), jnp.max(el), jnp.float32)


def _tc_prep(x, w, al, ar):
    return pl.pallas_call(
        _tc_prep_body,
        out_shape=(
            jax.ShapeDtypeStruct((N, D), jnp.float32),
            jax.ShapeDtypeStruct((N, 1), jnp.float32),
            jax.ShapeDtypeStruct((N, 1), jnp.float32),
            jax.ShapeDtypeStruct((1, L), jnp.float32),
        ),
    )(x, w, al, ar)


def _tc_mid_body(acc_ref, asum_ref, b_ref, g_ref, be_ref, w_ref, al_ref,
                 ar_ref, z_ref, el_ref, er_ref, em_ref):
    s = jnp.sum(asum_ref[...], axis=0)
    h = (acc_ref[0] + acc_ref[1]) / (s + 1e-9) + b_ref[...]
    mu = jnp.mean(h, axis=1, keepdims=True)
    var = jnp.mean((h - mu) ** 2, axis=1, keepdims=True)
    h = (h - mu) / jnp.sqrt(var + 1e-5) * g_ref[...] + be_ref[...]
    h = jnp.maximum(h, 0.0)
    z = jnp.dot(h, w_ref[...], preferred_element_type=jnp.float32)
    z_ref[...] = z
    el = jnp.sum(z * al_ref[...], axis=1, keepdims=True)
    el_ref[...] = el
    er_ref[...] = jnp.sum(z * ar_ref[...], axis=1, keepdims=True)
    em_ref[...] = jnp.full((1, L), jnp.max(el), jnp.float32)


def _tc_mid(acc, asum3, b, gamma, beta, w, al, ar):
    return pl.pallas_call(
        _tc_mid_body,
        out_shape=(
            jax.ShapeDtypeStruct((N, D), jnp.float32),
            jax.ShapeDtypeStruct((N, 1), jnp.float32),
            jax.ShapeDtypeStruct((N, 1), jnp.float32),
            jax.ShapeDtypeStruct((1, L), jnp.float32),
        ),
    )(acc, asum3, b, gamma, beta, w, al, ar)


def _tc_final_body(acc_ref, asum_ref, b_ref, out_ref):
    s = jnp.sum(asum_ref[...], axis=0)
    out_ref[...] = (acc_ref[0] + acc_ref[1]) / (s + 1e-9) + b_ref[...]


def _tc_final(acc, asum3, b):
    return pl.pallas_call(
        _tc_final_body,
        out_shape=jax.ShapeDtypeStruct((N, D), jnp.float32),
    )(acc, asum3, b)


def _sc_edge_body(src_hbm, dst4_hbm, el_hbm, er_hbm, emax_hbm, z_hbm,
                  zeros_hbm, zerosn_hbm,
                  acc_out, asum_out,
                  srcb, dstb, elb, erb, wbuf, rowbuf, emax_s, asumstg,
                  acc_sh, asum_sh, gsem0, gsem1, ssem0, ssem1):
    cid = lax.axis_index("c")
    sid = lax.axis_index("s")
    wid = cid * NS + sid
    ebase = wid * EPTP

    pltpu.sync_copy(emax_hbm, emax_s)

    @pl.when(sid == 0)
    def _zero_acc():
        pltpu.sync_copy(zeros_hbm, acc_sh)

    @pl.when(sid == 1)
    def _zero_asum():
        pltpu.sync_copy(zerosn_hbm, asum_sh)
    plsc.subcore_barrier()

    emaxv = emax_s[pl.ds(0, L)]
    lane = lax.iota(jnp.int32, L)
    gsems = (gsem0, gsem1)
    ssems = (ssem0, ssem1)

    def stage_block(b):
        bp = b % 2
        pltpu.sync_copy(src_hbm.at[pl.ds(ebase + b * BLKE, BLKE)],
                        srcb.at[bp])
        pltpu.sync_copy(dst4_hbm.at[wid, b], dstb.at[bp])

    def src_idx_full(cc):
        bp = (cc // BLK) % 2
        return srcb.at[bp, pl.ds((cc % BLK) * K, K)]

    def dst_idx(cc):
        bp = (cc // BLK) % 2
        return dstb.at[bp, cc % BLK]

    def issue_gathers(cc, p):
        pltpu.async_copy(z_hbm.at[src_idx_full(cc)], rowbuf.at[p], gsems[p])
        pltpu.async_copy(el_hbm.at[src_idx_full(cc)], elb.at[p], gsems[p])
        pltpu.async_copy(er_hbm.at[dst_idx(cc)], erb.at[p], gsems[p])

    def wait_gathers(cc, p):
        pltpu.make_async_copy(z_hbm.at[src_idx_full(cc)], rowbuf.at[p],
                              gsems[p]).wait()
        pltpu.make_async_copy(el_hbm.at[src_idx_full(cc)], elb.at[p],
                              gsems[p]).wait()
        pltpu.make_async_copy(er_hbm.at[dst_idx(cc)], erb.at[p],
                              gsems[p]).wait()

    def issue_scatters(cc, p):
        pltpu.async_copy(wbuf.at[p], asum_sh.at[dst_idx(cc)], ssems[p],
                         add=True)
        pltpu.async_copy(rowbuf.at[p], acc_sh.at[dst_idx(cc)], ssems[p],
                         add=True)

    def wait_scatters(cc, p):
        pltpu.make_async_copy(wbuf.at[p], asum_sh.at[dst_idx(cc)],
                              ssems[p]).wait()
        pltpu.make_async_copy(rowbuf.at[p], acc_sh.at[dst_idx(cc)],
                              ssems[p]).wait()

    def compute_scale(cc, p):
        offv = cc * K
        for v in range(K // L):
            els = elb[p, pl.ds(v * L, L)]
            erd = erb[p, pl.ds(v * L, L)]
            u = els + erd
            ev = jnp.maximum(u, 0.2 * u)
            t = erd + emaxv
            cv = jnp.maximum(t, 0.2 * t)
            w = jnp.exp(ev - cv)
            pos = lane + (offv + v * L)
            w = jnp.where(pos < EPT, w, 0.0)
            wbuf[p, pl.ds(v * L, L)] = w

        def _svb(vb, c2):
            wv = wbuf[p, pl.ds(vb * L, L)]
            for l in range(L):
                j = vb * L + l
                wj = wv[l]
                for k in range(D // L):
                    sl = pl.ds(k * L, L)
                    rowbuf[p, j, sl] = rowbuf[p, j, sl] * wj
            return c2
        lax.fori_loop(0, K // L, _svb, 0)

    stage_block(0)
    issue_gathers(0, 0)

    def _pair(i, carry):
        for half in (0, 1):
            cc = 2 * i + half
            p = half
            q = 1 - half

            @pl.when(cc >= 1)
            def _ws(cc=cc, q=q):
                wait_scatters(cc - 1, q)

            @pl.when(jnp.logical_and(cc + 1 < NCH, (cc + 1) % BLK == 0))
            def _sb(cc=cc):
                stage_block((cc + 1) // BLK)

            @pl.when(cc + 1 < NCH)
            def _ig(cc=cc, q=q):
                issue_gathers(cc + 1, q)

            wait_gathers(cc, p)
            compute_scale(cc, p)
            issue_scatters(cc, p)
        return carry
    lax.fori_loop(0, NCH // 2, _pair, 0)

    wait_scatters(NCH - 1, 1)
    plsc.subcore_barrier()

    @pl.when(sid == 0)
    def _wb():
        def _wbk(k5, c3):
            pltpu.sync_copy(asum_sh.at[pl.ds(k5 * 1000, 1000)], asumstg)
            pltpu.sync_copy(asumstg,
                            asum_out.at[pl.ds(cid * N + k5 * 1000, 1000)])
            return c3
        lax.fori_loop(0, N // 1000, _wbk, 0)
        pltpu.sync_copy(acc_sh, acc_out.at[cid])


_sc_edge = functools.partial(
    pl.kernel,
    out_type=(
        jax.ShapeDtypeStruct((NC, N, D), jnp.float32),
        jax.ShapeDtypeStruct((NC * N,), jnp.float32),
    ),
    mesh=plsc.VectorSubcoreMesh(core_axis_name="c", subcore_axis_name="s",
                                num_cores=NC, num_subcores=NS),
    compiler_params=pltpu.CompilerParams(needs_layout_passes=False),
    scratch_types=[
        pltpu.VMEM((2, BLKE), jnp.int32),
        pltpu.VMEM((2, BLK, K), jnp.int32),
        pltpu.VMEM((2, K), jnp.float32),
        pltpu.VMEM((2, K), jnp.float32),
        pltpu.VMEM((2, K), jnp.float32),
        pltpu.VMEM((2, K, D), jnp.float32),
        pltpu.VMEM((L,), jnp.float32),
        pltpu.VMEM((1000,), jnp.float32),
        pltpu.VMEM_SHARED((N, D), jnp.float32),
        pltpu.VMEM_SHARED((N,), jnp.float32),
        pltpu.SemaphoreType.DMA,
        pltpu.SemaphoreType.DMA,
        pltpu.SemaphoreType.DMA,
        pltpu.SemaphoreType.DMA,
    ],
)(_sc_edge_body)


def kernel(feat, edge_index, W0, al0, ar0, b0, gamma, beta, W1, al1, ar1, b1):
    ei = edge_index.reshape(2, NW, EPT)
    eip = jnp.pad(ei, ((0, 0), (0, 0), (0, EPTP - EPT)))
    src = eip[0].reshape(NW * EPTP)
    dst4 = eip[1].reshape(NW, NBLK, BLK, K)
    zeros = jnp.zeros((N, D), jnp.float32)
    zerosn = jnp.zeros((N,), jnp.float32)

    z0, el0, er0, em0 = _tc_prep(feat, W0, al0, ar0)
    acc0, asum0 = _sc_edge(src, dst4, el0.reshape(N), er0.reshape(N),
                           em0.reshape(L), z0, zeros, zerosn)
    z1, el1, er1, em1 = _tc_mid(acc0, asum0.reshape(NC, N, 1),
                                b0.reshape(1, D), gamma.reshape(1, D),
                                beta.reshape(1, D), W1, al1, ar1)
    acc1, asum1 = _sc_edge(src, dst4, el1.reshape(N), er1.reshape(N),
                           em1.reshape(L), z1, zeros, zerosn)
    return _tc_final(acc1, asum1.reshape(NC, N, 1), b1.reshape(1, D))
